# Initial kernel scaffold; baseline (speedup 1.0000x reference)
import jax
import jax.numpy as jnp
from jax.experimental import pallas as pl


def kernel(v_list, e_list, node_table, edge_table, W, b):
    raise NotImplementedError("write your pallas kernel here")


# speedup vs baseline: 2.0962x; 2.0962x over previous
import functools

import numpy as np
import jax
import jax.numpy as jnp
from jax import lax
from jax.experimental import pallas as pl
from jax.experimental.pallas import tpu as pltpu
from jax.experimental.pallas import tpu_sc as plsc

NODE_EMB = 64
EDGE_EMB = 64
D_MODEL = 128
LANES = 128
ROWS_PER_GRP = 4
GRP = LANES * ROWS_PER_GRP


def _pe_table(seq_len, d_model):
    pos = np.arange(seq_len, dtype=np.float32)[:, None]
    div = np.exp(np.arange(0, d_model, 2, dtype=np.float32)
                 * (-np.log(10000.0) / d_model))
    pe = np.zeros((seq_len, d_model), dtype=np.float32)
    pe[:, 0::2] = np.sin(pos * div)
    pe[:, 1::2] = np.cos(pos * div)
    return jnp.asarray(pe)


def _sc_gather(n_idx_rows, n_tokens):
    info = plsc.get_sparse_core_info()
    nc, ns = info.num_cores, info.num_subcores
    nw = nc * ns
    rows_per_w = n_idx_rows // nw
    grps = rows_per_w // ROWS_PER_GRP
    mesh = plsc.VectorSubcoreMesh(core_axis_name="c", subcore_axis_name="s")

    @functools.partial(
        pl.kernel, mesh=mesh,
        out_type=(jax.ShapeDtypeStruct((n_tokens, NODE_EMB), jnp.float32),
                  jax.ShapeDtypeStruct((n_tokens, EDGE_EMB), jnp.float32)),
        scratch_types=[
            pltpu.VMEM((ROWS_PER_GRP, LANES), jnp.int32),
            pltpu.VMEM((ROWS_PER_GRP, LANES), jnp.int32),
            pltpu.VMEM((GRP, NODE_EMB), jnp.float32),
            pltpu.VMEM((GRP, EDGE_EMB), jnp.float32),
            pltpu.SemaphoreType.DMA,
        ],
        compiler_params=pltpu.CompilerParams(use_tc_tiling_on_sc=False),
    )
    def gather_k(vidx_hbm, eidx_hbm, ntab_hbm, etab_hbm,
                 vout_hbm, eout_hbm, vidx, eidx, vrows, erows, sem):
        wid = lax.axis_index("s") * nc + lax.axis_index("c")
        row0 = wid * rows_per_w

        def body(g, carry):
            r = row0 + g * ROWS_PER_GRP
            pltpu.sync_copy(vidx_hbm.at[pl.ds(r, ROWS_PER_GRP)], vidx)
            pltpu.sync_copy(eidx_hbm.at[pl.ds(r, ROWS_PER_GRP)], eidx)
            cps = []
            for j in range(ROWS_PER_GRP):
                cps.append(pltpu.async_copy(
                    ntab_hbm.at[vidx.at[j]],
                    vrows.at[pl.ds(j * LANES, LANES)], sem))
                cps.append(pltpu.async_copy(
                    etab_hbm.at[eidx.at[j]],
                    erows.at[pl.ds(j * LANES, LANES)], sem))
            for c in cps:
                c.wait()
            tok = r * LANES
            pltpu.sync_copy(vrows, vout_hbm.at[pl.ds(tok, GRP)])
            pltpu.sync_copy(erows, eout_hbm.at[pl.ds(tok, GRP)])
            return carry

        lax.fori_loop(0, grps, body, 0)

    return gather_k


def _tc_body(v_ref, e_ref, wv_ref, we_ref, b_ref, pe_ref, h_ref, hp_ref):
    bb, seq, _ = v_ref.shape
    v = v_ref[...].reshape(bb * seq, NODE_EMB)
    e = e_ref[...].reshape(bb * seq, EDGE_EMB)
    h = jnp.dot(v, wv_ref[...], preferred_element_type=jnp.float32)
    h = h + jnp.dot(e, we_ref[...], preferred_element_type=jnp.float32)
    h = h + b_ref[...]
    h = h.reshape(bb, seq, D_MODEL)
    h_ref[...] = h
    hp_ref[...] = h + pe_ref[...][None]


def kernel(v_list, e_list, node_table, edge_table, W, b):
    B, L = v_list.shape
    n_tokens = B * L
    n_idx_rows = n_tokens // LANES

    v2d = v_list.reshape(n_idx_rows, LANES)
    e2d = e_list.reshape(n_idx_rows, LANES)

    v_emb, e_emb = _sc_gather(n_idx_rows, n_tokens)(
        v2d, e2d, node_table, edge_table)

    scale = np.sqrt(float(NODE_EMB))
    wv = (W[:, :NODE_EMB] * scale).T
    we = (W[:, NODE_EMB:] * scale).T
    b2 = b.reshape(1, D_MODEL)
    pe = _pe_table(L, D_MODEL)

    BB = 32
    grid = (B // BB,)
    h_emb, h_pos = pl.pallas_call(
        _tc_body,
        grid=grid,
        in_specs=[
            pl.BlockSpec((BB, L, NODE_EMB), lambda i: (i, 0, 0)),
            pl.BlockSpec((BB, L, EDGE_EMB), lambda i: (i, 0, 0)),
            pl.BlockSpec((NODE_EMB, D_MODEL), lambda i: (0, 0)),
            pl.BlockSpec((EDGE_EMB, D_MODEL), lambda i: (0, 0)),
            pl.BlockSpec((1, D_MODEL), lambda i: (0, 0)),
            pl.BlockSpec((L, D_MODEL), lambda i: (0, 0)),
        ],
        out_specs=[
            pl.BlockSpec((BB, L, D_MODEL), lambda i: (i, 0, 0)),
            pl.BlockSpec((BB, L, D_MODEL), lambda i: (i, 0, 0)),
        ],
        out_shape=[
            jax.ShapeDtypeStruct((B, L, D_MODEL), jnp.float32),
            jax.ShapeDtypeStruct((B, L, D_MODEL), jnp.float32),
        ],
    )(v_emb.reshape(B, L, NODE_EMB), e_emb.reshape(B, L, EDGE_EMB),
      wv, we, b2, pe)

    return (h_emb, h_pos)


# speedup vs baseline: 3.1965x; 1.5249x over previous
import functools

import numpy as np
import jax
import jax.numpy as jnp
from jax import lax
from jax.experimental import pallas as pl
from jax.experimental.pallas import tpu as pltpu
from jax.experimental.pallas import tpu_sc as plsc

NODE_EMB = 64
EDGE_EMB = 64
D_MODEL = 128
LANES = 128
ROWS_PER_GRP = 4
GRP = LANES * ROWS_PER_GRP


def _pe_table(seq_len, d_model):
    pos = np.arange(seq_len, dtype=np.float32)[:, None]
    div = np.exp(np.arange(0, d_model, 2, dtype=np.float32)
                 * (-np.log(10000.0) / d_model))
    pe = np.zeros((seq_len, d_model), dtype=np.float32)
    pe[:, 0::2] = np.sin(pos * div)
    pe[:, 1::2] = np.cos(pos * div)
    return jnp.asarray(pe)


def _sc_gather(n_idx_rows, n_tokens):
    info = plsc.get_sparse_core_info()
    nc, ns = info.num_cores, info.num_subcores
    nw = nc * ns
    rows_per_w = n_idx_rows // nw
    grps = rows_per_w // ROWS_PER_GRP
    mesh = plsc.VectorSubcoreMesh(core_axis_name="c", subcore_axis_name="s")

    @functools.partial(
        pl.kernel, mesh=mesh,
        out_type=jax.ShapeDtypeStruct((n_tokens, D_MODEL), jnp.float32),
        scratch_types=[
            pltpu.VMEM((ROWS_PER_GRP, LANES), jnp.int32),
            pltpu.VMEM((ROWS_PER_GRP, LANES), jnp.int32),
            pltpu.VMEM((GRP, NODE_EMB), jnp.float32),
            pltpu.VMEM((GRP, EDGE_EMB), jnp.float32),
            pltpu.SemaphoreType.DMA,
        ],
        compiler_params=pltpu.CompilerParams(use_tc_tiling_on_sc=False),
    )
    def gather_k(vidx_hbm, eidx_hbm, ntab_hbm, etab_hbm,
                 cat_hbm, vidx, eidx, vrows, erows, sem):
        wid = lax.axis_index("s") * nc + lax.axis_index("c")
        row0 = wid * rows_per_w

        def body(g, carry):
            r = row0 + g * ROWS_PER_GRP
            pltpu.sync_copy(vidx_hbm.at[pl.ds(r, ROWS_PER_GRP)], vidx)
            pltpu.sync_copy(eidx_hbm.at[pl.ds(r, ROWS_PER_GRP)], eidx)
            cps = []
            for j in range(ROWS_PER_GRP):
                cps.append(pltpu.async_copy(
                    ntab_hbm.at[vidx.at[j]],
                    vrows.at[pl.ds(j * LANES, LANES)], sem))
                cps.append(pltpu.async_copy(
                    etab_hbm.at[eidx.at[j]],
                    erows.at[pl.ds(j * LANES, LANES)], sem))
            for c in cps:
                c.wait()
            tok = r * LANES
            pltpu.sync_copy(
                vrows, cat_hbm.at[pl.ds(tok, GRP), pl.ds(0, NODE_EMB)])
            pltpu.sync_copy(
                erows, cat_hbm.at[pl.ds(tok, GRP), pl.ds(NODE_EMB, EDGE_EMB)])
            return carry

        lax.fori_loop(0, grps, body, 0)

    return gather_k


def _tc_body(cat_ref, w_ref, b_ref, pe_ref, h_ref, hp_ref):
    bb, seq, _ = cat_ref.shape
    x = cat_ref[...].reshape(bb * seq, D_MODEL)
    h = jnp.dot(x, w_ref[...], preferred_element_type=jnp.float32)
    h = h + b_ref[...]
    h = h.reshape(bb, seq, D_MODEL)
    h_ref[...] = h
    hp_ref[...] = h + pe_ref[...][None]


def kernel(v_list, e_list, node_table, edge_table, W, b):
    B, L = v_list.shape
    n_tokens = B * L
    n_idx_rows = n_tokens // LANES

    v2d = v_list.reshape(n_idx_rows, LANES)
    e2d = e_list.reshape(n_idx_rows, LANES)

    h_cat = _sc_gather(n_idx_rows, n_tokens)(
        v2d, e2d, node_table, edge_table)

    w2 = (W * np.sqrt(float(NODE_EMB))).T
    b2 = b.reshape(1, D_MODEL)
    pe = _pe_table(L, D_MODEL)

    BB = 32
    grid = (B // BB,)
    h_emb, h_pos = pl.pallas_call(
        _tc_body,
        grid=grid,
        in_specs=[
            pl.BlockSpec((BB, L, D_MODEL), lambda i: (i, 0, 0)),
            pl.BlockSpec((D_MODEL, D_MODEL), lambda i: (0, 0)),
            pl.BlockSpec((1, D_MODEL), lambda i: (0, 0)),
            pl.BlockSpec((L, D_MODEL), lambda i: (0, 0)),
        ],
        out_specs=[
            pl.BlockSpec((BB, L, D_MODEL), lambda i: (i, 0, 0)),
            pl.BlockSpec((BB, L, D_MODEL), lambda i: (i, 0, 0)),
        ],
        out_shape=[
            jax.ShapeDtypeStruct((B, L, D_MODEL), jnp.float32),
            jax.ShapeDtypeStruct((B, L, D_MODEL), jnp.float32),
        ],
    )(h_cat.reshape(B, L, D_MODEL), w2, b2, pe)

    return (h_emb, h_pos)


# speedup vs baseline: 3.2625x; 1.0207x over previous
import functools

import numpy as np
import jax
import jax.numpy as jnp
from jax import lax
from jax.experimental import pallas as pl
from jax.experimental.pallas import tpu as pltpu
from jax.experimental.pallas import tpu_sc as plsc

NODE_EMB = 64
EDGE_EMB = 64
D_MODEL = 128
LANES = 128
ROWS_PER_GRP = 4
GRP = LANES * ROWS_PER_GRP
N_CHUNKS = 2
BB = 32


def _pe_table(seq_len, d_model):
    pos = np.arange(seq_len, dtype=np.float32)[:, None]
    div = np.exp(np.arange(0, d_model, 2, dtype=np.float32)
                 * (-np.log(10000.0) / d_model))
    pe = np.zeros((seq_len, d_model), dtype=np.float32)
    pe[:, 0::2] = np.sin(pos * div)
    pe[:, 1::2] = np.cos(pos * div)
    return jnp.asarray(pe)


def _sc_gather(n_idx_rows, n_tokens):
    info = plsc.get_sparse_core_info()
    nc, ns = info.num_cores, info.num_subcores
    nw = nc * ns
    rows_per_w = n_idx_rows // nw
    grps = rows_per_w // ROWS_PER_GRP
    mesh = plsc.VectorSubcoreMesh(core_axis_name="c", subcore_axis_name="s")

    @functools.partial(
        pl.kernel, mesh=mesh,
        out_type=jax.ShapeDtypeStruct((n_tokens, D_MODEL), jnp.float32),
        scratch_types=[
            pltpu.VMEM((ROWS_PER_GRP, LANES), jnp.int32),
            pltpu.VMEM((ROWS_PER_GRP, LANES), jnp.int32),
            pltpu.VMEM((GRP, NODE_EMB), jnp.float32),
            pltpu.VMEM((GRP, EDGE_EMB), jnp.float32),
            pltpu.SemaphoreType.DMA,
        ],
        compiler_params=pltpu.CompilerParams(use_tc_tiling_on_sc=False),
    )
    def gather_k(vidx_hbm, eidx_hbm, ntab_hbm, etab_hbm,
                 cat_hbm, vidx, eidx, vrows, erows, sem):
        wid = lax.axis_index("s") * nc + lax.axis_index("c")
        row0 = wid * rows_per_w

        def body(g, carry):
            r = row0 + g * ROWS_PER_GRP
            pltpu.sync_copy(vidx_hbm.at[pl.ds(r, ROWS_PER_GRP)], vidx)
            pltpu.sync_copy(eidx_hbm.at[pl.ds(r, ROWS_PER_GRP)], eidx)
            cps = []
            for j in range(ROWS_PER_GRP):
                cps.append(pltpu.async_copy(
                    ntab_hbm.at[vidx.at[j]],
                    vrows.at[pl.ds(j * LANES, LANES)], sem))
                cps.append(pltpu.async_copy(
                    etab_hbm.at[eidx.at[j]],
                    erows.at[pl.ds(j * LANES, LANES)], sem))
            for c in cps:
                c.wait()
            tok = r * LANES
            pltpu.sync_copy(
                vrows, cat_hbm.at[pl.ds(tok, GRP), pl.ds(0, NODE_EMB)])
            pltpu.sync_copy(
                erows, cat_hbm.at[pl.ds(tok, GRP), pl.ds(NODE_EMB, EDGE_EMB)])
            return carry

        lax.fori_loop(0, grps, body, 0)

    return gather_k


def _tc_compute(cat_ref, w_ref, b_ref, pe_ref, h_ref, hp_ref):
    bb, seq, _ = cat_ref.shape
    x = cat_ref[...].reshape(bb * seq, D_MODEL)
    h = jnp.dot(x, w_ref[...], preferred_element_type=jnp.float32)
    h = h + b_ref[...]
    h = h.reshape(bb, seq, D_MODEL)
    h_ref[...] = h
    hp_ref[...] = h + pe_ref[...][None]


def _tc_body_first(cat_ref, w_ref, b_ref, pe_ref, h_ref, hp_ref):
    _tc_compute(cat_ref, w_ref, b_ref, pe_ref, h_ref, hp_ref)


def _tc_body_next(cat_ref, w_ref, b_ref, pe_ref, hin_ref, hpin_ref,
                  h_ref, hp_ref):
    del hin_ref, hpin_ref
    _tc_compute(cat_ref, w_ref, b_ref, pe_ref, h_ref, hp_ref)


def kernel(v_list, e_list, node_table, edge_table, W, b):
    B, L = v_list.shape
    n_tokens = B * L
    n_idx_rows = n_tokens // LANES

    v2d = v_list.reshape(n_idx_rows, LANES)
    e2d = e_list.reshape(n_idx_rows, LANES)

    rows_c = n_idx_rows // N_CHUNKS
    tok_c = n_tokens // N_CHUNKS
    batch_c = B // N_CHUNKS
    nblk_c = batch_c // BB

    gk = _sc_gather(rows_c, tok_c)
    cats = [
        gk(v2d[k * rows_c:(k + 1) * rows_c],
           e2d[k * rows_c:(k + 1) * rows_c],
           node_table, edge_table).reshape(batch_c, L, D_MODEL)
        for k in range(N_CHUNKS)
    ]

    w2 = (W * np.sqrt(float(NODE_EMB))).T
    b2 = b.reshape(1, D_MODEL)
    pe = _pe_table(L, D_MODEL)

    out_shape = [
        jax.ShapeDtypeStruct((B, L, D_MODEL), jnp.float32),
        jax.ShapeDtypeStruct((B, L, D_MODEL), jnp.float32),
    ]
    common_specs = [
        pl.BlockSpec((D_MODEL, D_MODEL), lambda i: (0, 0)),
        pl.BlockSpec((1, D_MODEL), lambda i: (0, 0)),
        pl.BlockSpec((L, D_MODEL), lambda i: (0, 0)),
    ]

    h_emb, h_pos = pl.pallas_call(
        _tc_body_first,
        grid=(nblk_c,),
        in_specs=[pl.BlockSpec((BB, L, D_MODEL), lambda i: (i, 0, 0))]
        + common_specs,
        out_specs=[
            pl.BlockSpec((BB, L, D_MODEL), lambda i: (i, 0, 0)),
            pl.BlockSpec((BB, L, D_MODEL), lambda i: (i, 0, 0)),
        ],
        out_shape=out_shape,
    )(cats[0], w2, b2, pe)

    for k in range(1, N_CHUNKS):
        off = k * nblk_c
        h_emb, h_pos = pl.pallas_call(
            _tc_body_next,
            grid=(nblk_c,),
            in_specs=[pl.BlockSpec((BB, L, D_MODEL), lambda i: (i, 0, 0))]
            + common_specs
            + [pl.BlockSpec(memory_space=pl.ANY),
               pl.BlockSpec(memory_space=pl.ANY)],
            out_specs=[
                pl.BlockSpec((BB, L, D_MODEL),
                             lambda i, off=off: (i + off, 0, 0)),
                pl.BlockSpec((BB, L, D_MODEL),
                             lambda i, off=off: (i + off, 0, 0)),
            ],
            out_shape=out_shape,
            input_output_aliases={4: 0, 5: 1},
        )(cats[k], w2, b2, pe, h_emb, h_pos)

    return (h_emb, h_pos)


# speedup vs baseline: 3.5882x; 1.0998x over previous
import functools

import numpy as np
import jax
import jax.numpy as jnp
from jax import lax
from jax.experimental import pallas as pl
from jax.experimental.pallas import tpu as pltpu
from jax.experimental.pallas import tpu_sc as plsc

NODE_EMB = 64
EDGE_EMB = 64
D_MODEL = 128
LANES = 128
ROWS_PER_GRP = 4
GRP = LANES * ROWS_PER_GRP
N_CHUNKS = 2
BB = 32


def _pe_table(seq_len, d_model):
    pos = np.arange(seq_len, dtype=np.float32)[:, None]
    div = np.exp(np.arange(0, d_model, 2, dtype=np.float32)
                 * (-np.log(10000.0) / d_model))
    pe = np.zeros((seq_len, d_model), dtype=np.float32)
    pe[:, 0::2] = np.sin(pos * div)
    pe[:, 1::2] = np.cos(pos * div)
    return jnp.asarray(pe)


def _sc_gather(n_idx_rows, n_tokens):
    info = plsc.get_sparse_core_info()
    nc, ns = info.num_cores, info.num_subcores
    nw = nc * ns
    rows_per_w = n_idx_rows // nw
    grps = rows_per_w // ROWS_PER_GRP
    mesh = plsc.VectorSubcoreMesh(core_axis_name="c", subcore_axis_name="s")

    @functools.partial(
        pl.kernel, mesh=mesh,
        out_type=jax.ShapeDtypeStruct((n_tokens, D_MODEL), jnp.float32),
        scratch_types=[
            pltpu.VMEM((ROWS_PER_GRP, LANES), jnp.int32),
            pltpu.VMEM((ROWS_PER_GRP, LANES), jnp.int32),
            pltpu.VMEM((GRP, NODE_EMB), jnp.float32),
            pltpu.VMEM((GRP, EDGE_EMB), jnp.float32),
            pltpu.SemaphoreType.DMA,
        ],
        compiler_params=pltpu.CompilerParams(use_tc_tiling_on_sc=False),
    )
    def gather_k(vidx_hbm, eidx_hbm, ntab_hbm, etab_hbm,
                 cat_hbm, vidx, eidx, vrows, erows, sem):
        wid = lax.axis_index("s") * nc + lax.axis_index("c")
        row0 = wid * rows_per_w

        def body(g, carry):
            r = row0 + g * ROWS_PER_GRP
            pltpu.sync_copy(vidx_hbm.at[pl.ds(r, ROWS_PER_GRP)], vidx)
            pltpu.sync_copy(eidx_hbm.at[pl.ds(r, ROWS_PER_GRP)], eidx)
            cps = []
            for j in range(ROWS_PER_GRP):
                cps.append(pltpu.async_copy(
                    ntab_hbm.at[vidx.at[j]],
                    vrows.at[pl.ds(j * LANES, LANES)], sem))
                cps.append(pltpu.async_copy(
                    etab_hbm.at[eidx.at[j]],
                    erows.at[pl.ds(j * LANES, LANES)], sem))
            for c in cps:
                c.wait()
            tok = r * LANES
            pltpu.sync_copy(
                vrows, cat_hbm.at[pl.ds(tok, GRP), pl.ds(0, NODE_EMB)])
            pltpu.sync_copy(
                erows, cat_hbm.at[pl.ds(tok, GRP), pl.ds(NODE_EMB, EDGE_EMB)])
            return carry

        lax.fori_loop(0, grps, body, 0)

    return gather_k


_TRANSP_CB = 1024


def _transp_body(a_ref, b_ref, o_ref):
    o_ref[:, 0:NODE_EMB] = a_ref[...].T
    o_ref[:, NODE_EMB:] = b_ref[...].T


def _relayout_table(table):
    V = table.shape[0]
    grid_n = (V + 2 * _TRANSP_CB - 1) // (2 * _TRANSP_CB)
    max_blk = (V + _TRANSP_CB - 1) // _TRANSP_CB - 1
    tT = table.T
    packed = pl.pallas_call(
        _transp_body,
        grid=(grid_n,),
        in_specs=[
            pl.BlockSpec((NODE_EMB, _TRANSP_CB), lambda i: (0, 2 * i)),
            pl.BlockSpec((NODE_EMB, _TRANSP_CB),
                         lambda i, m=max_blk: (0, jnp.minimum(2 * i + 1, m))),
        ],
        out_specs=pl.BlockSpec((_TRANSP_CB, 2 * NODE_EMB), lambda i: (i, 0)),
        out_shape=jax.ShapeDtypeStruct(
            (grid_n * _TRANSP_CB, 2 * NODE_EMB), jnp.float32),
    )(tT, tT)
    return packed.reshape(grid_n * _TRANSP_CB * 2, NODE_EMB)


def _remap_idx(idx):
    r = idx & (2 * _TRANSP_CB - 1)
    return (idx - r) + ((r & (_TRANSP_CB - 1)) << 1) + (r >> 10)


def _tc_compute(cat_ref, w_ref, b_ref, pe_ref, h_ref, hp_ref):
    bb, seq, _ = cat_ref.shape
    x = cat_ref[...].reshape(bb * seq, D_MODEL)
    h = jnp.dot(x, w_ref[...], preferred_element_type=jnp.float32)
    h = h + b_ref[...]
    h = h.reshape(bb, seq, D_MODEL)
    h_ref[...] = h
    hp_ref[...] = h + pe_ref[...][None]


def _tc_body_first(cat_ref, w_ref, b_ref, pe_ref, h_ref, hp_ref):
    _tc_compute(cat_ref, w_ref, b_ref, pe_ref, h_ref, hp_ref)


def _tc_body_next(cat_ref, w_ref, b_ref, pe_ref, hin_ref, hpin_ref,
                  h_ref, hp_ref):
    del hin_ref, hpin_ref
    _tc_compute(cat_ref, w_ref, b_ref, pe_ref, h_ref, hp_ref)


def kernel(v_list, e_list, node_table, edge_table, W, b):
    B, L = v_list.shape
    n_tokens = B * L
    n_idx_rows = n_tokens // LANES

    nt_lin = _relayout_table(node_table)
    et_lin = _relayout_table(edge_table)

    v2d = _remap_idx(v_list.reshape(n_idx_rows, LANES))
    e2d = _remap_idx(e_list.reshape(n_idx_rows, LANES))

    rows_c = n_idx_rows // N_CHUNKS
    tok_c = n_tokens // N_CHUNKS
    batch_c = B // N_CHUNKS
    nblk_c = batch_c // BB

    gk = _sc_gather(rows_c, tok_c)
    cats = [
        gk(v2d[k * rows_c:(k + 1) * rows_c],
           e2d[k * rows_c:(k + 1) * rows_c],
           nt_lin, et_lin).reshape(batch_c, L, D_MODEL)
        for k in range(N_CHUNKS)
    ]

    w2 = (W * np.sqrt(float(NODE_EMB))).T
    b2 = b.reshape(1, D_MODEL)
    pe = _pe_table(L, D_MODEL)

    out_shape = [
        jax.ShapeDtypeStruct((B, L, D_MODEL), jnp.float32),
        jax.ShapeDtypeStruct((B, L, D_MODEL), jnp.float32),
    ]
    common_specs = [
        pl.BlockSpec((D_MODEL, D_MODEL), lambda i: (0, 0)),
        pl.BlockSpec((1, D_MODEL), lambda i: (0, 0)),
        pl.BlockSpec((L, D_MODEL), lambda i: (0, 0)),
    ]

    h_emb, h_pos = pl.pallas_call(
        _tc_body_first,
        grid=(nblk_c,),
        in_specs=[pl.BlockSpec((BB, L, D_MODEL), lambda i: (i, 0, 0))]
        + common_specs,
        out_specs=[
            pl.BlockSpec((BB, L, D_MODEL), lambda i: (i, 0, 0)),
            pl.BlockSpec((BB, L, D_MODEL), lambda i: (i, 0, 0)),
        ],
        out_shape=out_shape,
    )(cats[0], w2, b2, pe)

    for k in range(1, N_CHUNKS):
        off = k * nblk_c
        h_emb, h_pos = pl.pallas_call(
            _tc_body_next,
            grid=(nblk_c,),
            in_specs=[pl.BlockSpec((BB, L, D_MODEL), lambda i: (i, 0, 0))]
            + common_specs
            + [pl.BlockSpec(memory_space=pl.ANY),
               pl.BlockSpec(memory_space=pl.ANY)],
            out_specs=[
                pl.BlockSpec((BB, L, D_MODEL),
                             lambda i, off=off: (i + off, 0, 0)),
                pl.BlockSpec((BB, L, D_MODEL),
                             lambda i, off=off: (i + off, 0, 0)),
            ],
            out_shape=out_shape,
            input_output_aliases={4: 0, 5: 1},
        )(cats[k], w2, b2, pe, h_emb, h_pos)

    return (h_emb, h_pos)
